# Initial kernel scaffold; baseline (speedup 1.0000x reference)
import jax
import jax.numpy as jnp
from jax.experimental import pallas as pl


def kernel(x, edge_index, W1, b1, W2, b2, W3, b3, W4, b4, W5, b5):
    raise NotImplementedError("write your pallas kernel here")


# speedup vs baseline: 11.0293x; 11.0293x over previous
import functools

import jax
import jax.numpy as jnp
from jax import lax
from jax.experimental import pallas as pl
from jax.experimental.pallas import tpu as pltpu
from jax.experimental.pallas import tpu_sc as plsc

N = 10000
NPAD = 10240
E = 320000
CH = 80
NC, NS = 2, 16
ROWS_T = NPAD // NS
NCH_E = E // (NC * NS * CH)
NCH_C = E // (NS * CH)
ZR = 128

_mesh = plsc.VectorSubcoreMesh(core_axis_name="c", subcore_axis_name="s",
                               num_cores=NC)


def _deg_body(dst3, zeros, out, idx_v, ones_v, zb_v, acc):
    c = lax.axis_index("c")
    s = lax.axis_index("s")
    w = c * NS + s
    pltpu.sync_copy(zeros, zb_v)
    pltpu.sync_copy(zb_v, acc.at[pl.ds(s * ROWS_T, ROWS_T)])

    def fill(i, _):
        ones_v[pl.ds(i * 16, 16)] = jnp.ones((16,), jnp.float32)
        return 0
    lax.fori_loop(0, CH // 16, fill, 0)

    pltpu.sync_copy(dst3.at[w], idx_v)
    plsc.subcore_barrier()

    def body(j, _):
        pltpu.sync_copy(ones_v, acc.at[idx_v.at[j]], add=True)
        return 0
    lax.fori_loop(0, NCH_E, body, 0)
    plsc.subcore_barrier()

    pltpu.sync_copy(acc.at[pl.ds(s * ROWS_T, ROWS_T)], zb_v)
    pltpu.sync_copy(zb_v, out.at[c, pl.ds(s * ROWS_T, ROWS_T)])


def _sc_degree(dst3, zeros):
    return pl.kernel(
        _deg_body,
        out_type=jax.ShapeDtypeStruct((NC, NPAD), jnp.float32),
        mesh=_mesh,
        scratch_types=[
            pltpu.VMEM((NCH_E, CH), jnp.int32),
            pltpu.VMEM((CH,), jnp.float32),
            pltpu.VMEM((ROWS_T,), jnp.float32),
            pltpu.VMEM_SHARED((NPAD,), jnp.float32),
        ],
    )(dst3, zeros)


def _agg_body(dc, ks, u, src3, dst3, zeros, out,
              idxs_v, idxd_v, gb_v, zb_v, acc):
    c = lax.axis_index("c")
    s = lax.axis_index("s")
    nch = NCH_C if ks > 1 else NCH_E
    w = s if ks > 1 else c * NS + s
    spc = max(ks // NC, 1)

    pltpu.sync_copy(zeros, zb_v)
    pltpu.sync_copy(dst3.at[w], idxd_v)

    for p in range(spc):
        strip = c * spc + p
        for r in range(ROWS_T // ZR):
            pltpu.sync_copy(zb_v, acc.at[pl.ds(s * ROWS_T + r * ZR, ZR)])

        pltpu.sync_copy(src3.at[w], idxs_v)
        if ks > 1:
            def trow(j, _):
                for uo in range(CH // 16):
                    v = idxs_v[j, pl.ds(uo * 16, 16)]
                    idxs_v[j, pl.ds(uo * 16, 16)] = v * ks + strip
                return 0
            lax.fori_loop(0, nch, trow, 0)

        plsc.subcore_barrier()

        def chunk(j, _):
            pltpu.sync_copy(u.at[idxs_v.at[j]], gb_v)
            pltpu.sync_copy(gb_v, acc.at[idxd_v.at[j]], add=True)
            return 0
        lax.fori_loop(0, nch, chunk, 0)
        plsc.subcore_barrier()

        for r in range(ROWS_T // ZR):
            pltpu.sync_copy(acc.at[pl.ds(s * ROWS_T + r * ZR, ZR)], zb_v)
            pltpu.sync_copy(zb_v, out.at[strip, pl.ds(s * ROWS_T + r * ZR, ZR)])


def _sc_aggregate(u, src, dst, d, ks):
    dc = d // max(ks, 1)
    nch = NCH_C if ks > 1 else NCH_E
    nw = NS if ks > 1 else NC * NS
    src3 = src.reshape(nw, nch, CH)
    dst3 = dst.reshape(nw, nch, CH)
    nstrip = max(ks, NC)
    uv = u.reshape(ks * N, dc)
    zeros = jnp.zeros((ZR, dc), jnp.float32)
    body = functools.partial(_agg_body, dc, ks)
    out = pl.kernel(
        body,
        out_type=jax.ShapeDtypeStruct((nstrip, NPAD, dc), jnp.float32),
        mesh=_mesh,
        compiler_params=pltpu.CompilerParams(use_tc_tiling_on_sc=False),
        scratch_types=[
            pltpu.VMEM((nch, CH), jnp.int32),
            pltpu.VMEM((nch, CH), jnp.int32),
            pltpu.VMEM((CH, dc), jnp.float32),
            pltpu.VMEM((ZR, dc), jnp.float32),
            pltpu.VMEM_SHARED((NPAD, dc), jnp.float32),
        ],
    )(uv, src3, dst3, zeros)
    return out[:, :N, :]


R = 2000


def _rep(shape):
    return pl.BlockSpec(shape, lambda i: tuple(0 for _ in shape))


def _tc_prolog(degT, x):
    def body(deg_ref, x_ref, dinv_ref, u1_ref):
        deg = deg_ref[:, 0:1] + deg_ref[:, 1:2] + 1.0
        dinv = lax.rsqrt(deg)
        dinv_ref[...] = dinv
        u1_ref[...] = x_ref[...] * dinv
    return pl.pallas_call(
        body,
        out_shape=(jax.ShapeDtypeStruct((N, 1), jnp.float32),
                   jax.ShapeDtypeStruct((N, 128), jnp.float32)),
    )(degT, x)


def _tc_stage2(S1, U1, dinv, W1, b1):
    def body(s_ref, u_ref, dv_ref, w_ref, b_ref, out_ref):
        dv = dv_ref[...]
        y = (jnp.concatenate([s_ref[0], s_ref[1]], axis=1) + u_ref[...]) * dv
        h = jnp.dot(y, w_ref[...], preferred_element_type=jnp.float32)
        h = jnp.maximum(h + b_ref[...], 0.0)
        out_ref[...] = h * dv
    return pl.pallas_call(
        body,
        grid=(N // R,),
        in_specs=[pl.BlockSpec((2, R, 64), lambda i: (0, i, 0)),
                  pl.BlockSpec((R, 128), lambda i: (i, 0)),
                  pl.BlockSpec((R, 1), lambda i: (i, 0)),
                  _rep((128, 256)), _rep((1, 256))],
        out_specs=pl.BlockSpec((R, 256), lambda i: (i, 0)),
        out_shape=jax.ShapeDtypeStruct((N, 256), jnp.float32),
    )(S1, U1, dinv, W1, b1.reshape(1, -1))


def _tc_stage3(S2, U2, dinv, W2, b2, W3):
    def body(s_ref, u_ref, dv_ref, w2_ref, b2_ref, w3_ref, out_ref):
        dv = dv_ref[...]
        cat = jnp.concatenate([s_ref[k] for k in range(4)], axis=1)
        y = (cat + u_ref[...]) * dv
        h = jnp.dot(y, w2_ref[...], preferred_element_type=jnp.float32)
        h = jnp.maximum(h + b2_ref[...], 0.0)
        t = jnp.dot(h, w3_ref[...], preferred_element_type=jnp.float32)
        out_ref[...] = t * dv
    return pl.pallas_call(
        body,
        grid=(N // R,),
        in_specs=[pl.BlockSpec((4, R, 64), lambda i: (0, i, 0)),
                  pl.BlockSpec((R, 256), lambda i: (i, 0)),
                  pl.BlockSpec((R, 1), lambda i: (i, 0)),
                  _rep((256, 512)), _rep((1, 512)), _rep((512, 256))],
        out_specs=pl.BlockSpec((R, 256), lambda i: (i, 0)),
        out_shape=jax.ShapeDtypeStruct((N, 256), jnp.float32),
    )(S2, U2, dinv, W2, b2.reshape(1, -1), W3)


def _tc_stage4(S3, U3, dinv, b3, W4):
    def body(s_ref, u_ref, dv_ref, b3_ref, w4_ref, out_ref):
        dv = dv_ref[...]
        cat = jnp.concatenate([s_ref[k] for k in range(4)], axis=1)
        y = (cat + u_ref[...]) * dv
        h = jnp.maximum(y + b3_ref[...], 0.0)
        t = jnp.dot(h, w4_ref[...], preferred_element_type=jnp.float32)
        out_ref[...] = t * dv
    return pl.pallas_call(
        body,
        grid=(N // R,),
        in_specs=[pl.BlockSpec((4, R, 64), lambda i: (0, i, 0)),
                  pl.BlockSpec((R, 256), lambda i: (i, 0)),
                  pl.BlockSpec((R, 1), lambda i: (i, 0)),
                  _rep((1, 256)), _rep((256, 64))],
        out_specs=pl.BlockSpec((R, 64), lambda i: (i, 0)),
        out_shape=jax.ShapeDtypeStruct((N, 64), jnp.float32),
    )(S3, U3, dinv, b3.reshape(1, -1), W4)


def _tc_stage5(S4, U4, dinv, b4, W5):
    def body(s_ref, u_ref, dv_ref, b4_ref, w5_ref, out_ref):
        dv = dv_ref[...]
        y = (jnp.concatenate([s_ref[0], s_ref[1]], axis=1) + u_ref[...]) * dv
        h = jnp.maximum(y + b4_ref[...], 0.0)
        t = jnp.dot(h, w5_ref[...], preferred_element_type=jnp.float32)
        out_ref[...] = t * dv
    return pl.pallas_call(
        body,
        grid=(N // R,),
        in_specs=[pl.BlockSpec((2, R, 32), lambda i: (0, i, 0)),
                  pl.BlockSpec((R, 64), lambda i: (i, 0)),
                  pl.BlockSpec((R, 1), lambda i: (i, 0)),
                  _rep((1, 64)), _rep((64, 2))],
        out_specs=pl.BlockSpec((R, 2), lambda i: (i, 0)),
        out_shape=jax.ShapeDtypeStruct((N, 2), jnp.float32),
    )(S4, U4, dinv, b4.reshape(1, -1), W5)


def _tc_final(S5, U5, dinv, b5):
    def body(s_ref, u_ref, dv_ref, b5_ref, out_ref):
        y = (s_ref[0] + s_ref[1] + u_ref[...]) * dv_ref[...] + b5_ref[...]
        m = jnp.max(y, axis=1, keepdims=True)
        z = y - m
        lse = jnp.log(jnp.sum(jnp.exp(z), axis=1, keepdims=True))
        out_ref[...] = z - lse
    return pl.pallas_call(
        body,
        out_shape=jax.ShapeDtypeStruct((N, 2), jnp.float32),
    )(S5, U5, dinv, b5.reshape(1, -1))


def kernel(x, edge_index, W1, b1, W2, b2, W3, b3, W4, b4, W5, b5):
    ei = edge_index.astype(jnp.int32)
    src = ei[0]
    dst = ei[1]
    zdeg = jnp.zeros((ROWS_T,), jnp.float32)
    dst3e = dst.reshape(NC * NS, NCH_E, CH)

    deg2 = _sc_degree(dst3e, zdeg)[:, :N]
    dinv, U1 = _tc_prolog(deg2.T, x)
    S1 = _sc_aggregate(U1, src, dst, 128, 2)
    U2 = _tc_stage2(S1, U1, dinv, W1, b1)
    S2 = _sc_aggregate(U2, src, dst, 256, 4)
    U3 = _tc_stage3(S2, U2, dinv, W2, b2, W3)
    S3 = _sc_aggregate(U3, src, dst, 256, 4)
    U4 = _tc_stage4(S3, U3, dinv, b3, W4)
    S4 = _sc_aggregate(U4, src, dst, 64, 2)
    U5 = _tc_stage5(S4, U4, dinv, b4, W5)
    S5 = _sc_aggregate(U5, src, dst, 2, 1)
    return _tc_final(S5, U5, dinv, b5)


# speedup vs baseline: 23.0166x; 2.0869x over previous
import functools

import jax
import jax.numpy as jnp
from jax import lax
from jax.experimental import pallas as pl
from jax.experimental.pallas import tpu as pltpu
from jax.experimental.pallas import tpu_sc as plsc

N = 10000
NPAD = 10240
E = 320000
CH = 80
CHE = 125
NBUF = 5
NC, NS = 2, 16
ROWS_T = NPAD // NS
NCH_E = E // (NC * NS * CHE)
NCH_C = E // (NS * CH)
ZR = 128

_mesh = plsc.VectorSubcoreMesh(core_axis_name="c", subcore_axis_name="s",
                               num_cores=NC)


def _deg_body(dst3, zeros, out, idx_v, ones_v, zb_v, acc, *dsems):
    c = lax.axis_index("c")
    s = lax.axis_index("s")
    w = c * NS + s
    pltpu.sync_copy(zeros, zb_v)
    pltpu.sync_copy(zb_v, acc.at[pl.ds(s * ROWS_T, ROWS_T)])

    def fill(i, _):
        ones_v[pl.ds(i * 16, 16)] = jnp.ones((16,), jnp.float32)
        return 0
    lax.fori_loop(0, 128 // 16, fill, 0)

    pltpu.sync_copy(dst3.at[w], idx_v)
    plsc.subcore_barrier()

    nsem = 2 * NBUF

    def grp(g, _):
        for b in range(nsem):
            j = g * nsem + b

            @pl.when(g > 0)
            def _():
                pltpu.make_async_copy(
                    ones_v.at[pl.ds(0, CHE)], acc.at[idx_v.at[0]],
                    dsems[b]).wait()
            pltpu.async_copy(ones_v.at[pl.ds(0, CHE)], acc.at[idx_v.at[j]],
                             dsems[b], add=True)
        return 0
    lax.fori_loop(0, NCH_E // nsem, grp, 0)
    for b in range(nsem):
        pltpu.make_async_copy(ones_v.at[pl.ds(0, CHE)], acc.at[idx_v.at[0]],
                              dsems[b]).wait()
    plsc.subcore_barrier()

    pltpu.sync_copy(acc.at[pl.ds(s * ROWS_T, ROWS_T)],
                    out.at[c, pl.ds(s * ROWS_T, ROWS_T)])


def _sc_degree(dst3, zeros):
    return pl.kernel(
        _deg_body,
        out_type=jax.ShapeDtypeStruct((NC, NPAD), jnp.float32),
        mesh=_mesh,
        scratch_types=[
            pltpu.VMEM((NCH_E, CHE), jnp.int32),
            pltpu.VMEM((128,), jnp.float32),
            pltpu.VMEM((ROWS_T,), jnp.float32),
            pltpu.VMEM_SHARED((NPAD,), jnp.float32),
        ] + [pltpu.SemaphoreType.DMA] * (2 * NBUF),
    )(dst3, zeros)


def _agg_body(dc, ks, u, src3, dst3, zeros, out,
              idxs_v, idxd_v, gb_v, zb_v, acc, *sems):
    c = lax.axis_index("c")
    s = lax.axis_index("s")
    nch = NCH_C if ks > 1 else NCH_E
    ch = CH if ks > 1 else CHE
    w = s if ks > 1 else c * NS + s
    spc = max(ks // NC, 1)
    gsems = sems[:NBUF]
    ssems = sems[NBUF:]

    pltpu.sync_copy(zeros, zb_v)
    pltpu.sync_copy(dst3.at[w], idxd_v)

    def swait(i):
        pltpu.make_async_copy(gb_v.at[i], acc.at[idxd_v.at[0]],
                              ssems[i]).wait()

    def gwait(i, j):
        pltpu.make_async_copy(u.at[idxs_v.at[j]], gb_v.at[i],
                              gsems[i]).wait()

    for p_strip in range(spc):
        strip = c * spc + p_strip
        for r in range(ROWS_T // ZR):
            pltpu.sync_copy(zb_v, acc.at[pl.ds(s * ROWS_T + r * ZR, ZR)])

        pltpu.sync_copy(src3.at[w], idxs_v)
        if ks > 1:
            def trow(j, _):
                for uo in range(ch // 16):
                    v = idxs_v[j, pl.ds(uo * 16, 16)]
                    idxs_v[j, pl.ds(uo * 16, 16)] = v * ks + strip
                return 0
            lax.fori_loop(0, nch, trow, 0)

        plsc.subcore_barrier()

        for b in range(NBUF):
            pltpu.async_copy(u.at[idxs_v.at[b]], gb_v.at[b], gsems[b])

        def group(g, _):
            for b in range(NBUF):
                j = g * NBUF + b
                gwait(b, j)
                pltpu.async_copy(gb_v.at[b], acc.at[idxd_v.at[j]],
                                 ssems[b], add=True)
            for b in range(NBUF):
                jn = (g + 1) * NBUF + b
                swait(b)

                @pl.when(jn < nch)
                def _():
                    pltpu.async_copy(u.at[idxs_v.at[jn]], gb_v.at[b],
                                     gsems[b])
            return 0
        lax.fori_loop(0, nch // NBUF, group, 0)
        plsc.subcore_barrier()

        for r in range(ROWS_T // ZR):
            pltpu.sync_copy(acc.at[pl.ds(s * ROWS_T + r * ZR, ZR)],
                            out.at[strip, pl.ds(s * ROWS_T + r * ZR, ZR)])


def _sc_aggregate(u, src, dst, d, ks):
    dc = d // max(ks, 1)
    nch = NCH_C if ks > 1 else NCH_E
    ch = CH if ks > 1 else CHE
    nw = NS if ks > 1 else NC * NS
    src3 = src.reshape(nw, nch, ch)
    dst3 = dst.reshape(nw, nch, ch)
    nstrip = max(ks, NC)
    uv = u.reshape(ks * N, dc)
    zeros = jnp.zeros((ZR, dc), jnp.float32)
    body = functools.partial(_agg_body, dc, ks)
    out = pl.kernel(
        body,
        out_type=jax.ShapeDtypeStruct((nstrip, NPAD, dc), jnp.float32),
        mesh=_mesh,
        compiler_params=pltpu.CompilerParams(use_tc_tiling_on_sc=False),
        scratch_types=[
            pltpu.VMEM((nch, ch), jnp.int32),
            pltpu.VMEM((nch, ch), jnp.int32),
            pltpu.VMEM((NBUF, ch, dc), jnp.float32),
            pltpu.VMEM((ZR, dc), jnp.float32),
            pltpu.VMEM_SHARED((NPAD, dc), jnp.float32),
        ] + [pltpu.SemaphoreType.DMA] * (2 * NBUF),
    )(uv, src3, dst3, zeros)
    return out[:, :N, :]


R = 2000


def _rep(shape):
    return pl.BlockSpec(shape, lambda i: tuple(0 for _ in shape))


def _tc_prolog(degT, x):
    def body(deg_ref, x_ref, dinv_ref, u1_ref):
        deg = deg_ref[:, 0:1] + deg_ref[:, 1:2] + 1.0
        dinv = lax.rsqrt(deg)
        dinv_ref[...] = dinv
        u1_ref[...] = x_ref[...] * dinv
    return pl.pallas_call(
        body,
        out_shape=(jax.ShapeDtypeStruct((N, 1), jnp.float32),
                   jax.ShapeDtypeStruct((N, 128), jnp.float32)),
    )(degT, x)


def _tc_stage2(S1, U1, dinv, W1, b1):
    def body(s_ref, u_ref, dv_ref, w_ref, b_ref, out_ref):
        dv = dv_ref[...]
        cat = jnp.concatenate([s_ref[k] for k in range(4)], axis=1)
        y = (cat + u_ref[...]) * dv
        h = jnp.dot(y, w_ref[...], preferred_element_type=jnp.float32)
        h = jnp.maximum(h + b_ref[...], 0.0)
        out_ref[...] = h * dv
    return pl.pallas_call(
        body,
        grid=(N // R,),
        in_specs=[pl.BlockSpec((4, R, 32), lambda i: (0, i, 0)),
                  pl.BlockSpec((R, 128), lambda i: (i, 0)),
                  pl.BlockSpec((R, 1), lambda i: (i, 0)),
                  _rep((128, 256)), _rep((1, 256))],
        out_specs=pl.BlockSpec((R, 256), lambda i: (i, 0)),
        out_shape=jax.ShapeDtypeStruct((N, 256), jnp.float32),
    )(S1, U1, dinv, W1, b1.reshape(1, -1))


def _tc_stage3(S2, U2, dinv, W2, b2, W3):
    def body(s_ref, u_ref, dv_ref, w2_ref, b2_ref, w3_ref, out_ref):
        dv = dv_ref[...]
        cat = jnp.concatenate([s_ref[k] for k in range(4)], axis=1)
        y = (cat + u_ref[...]) * dv
        h = jnp.dot(y, w2_ref[...], preferred_element_type=jnp.float32)
        h = jnp.maximum(h + b2_ref[...], 0.0)
        t = jnp.dot(h, w3_ref[...], preferred_element_type=jnp.float32)
        out_ref[...] = t * dv
    return pl.pallas_call(
        body,
        grid=(N // R,),
        in_specs=[pl.BlockSpec((4, R, 64), lambda i: (0, i, 0)),
                  pl.BlockSpec((R, 256), lambda i: (i, 0)),
                  pl.BlockSpec((R, 1), lambda i: (i, 0)),
                  _rep((256, 512)), _rep((1, 512)), _rep((512, 256))],
        out_specs=pl.BlockSpec((R, 256), lambda i: (i, 0)),
        out_shape=jax.ShapeDtypeStruct((N, 256), jnp.float32),
    )(S2, U2, dinv, W2, b2.reshape(1, -1), W3)


def _tc_stage4(S3, U3, dinv, b3, W4):
    def body(s_ref, u_ref, dv_ref, b3_ref, w4_ref, out_ref):
        dv = dv_ref[...]
        cat = jnp.concatenate([s_ref[k] for k in range(4)], axis=1)
        y = (cat + u_ref[...]) * dv
        h = jnp.maximum(y + b3_ref[...], 0.0)
        t = jnp.dot(h, w4_ref[...], preferred_element_type=jnp.float32)
        out_ref[...] = t * dv
    return pl.pallas_call(
        body,
        grid=(N // R,),
        in_specs=[pl.BlockSpec((4, R, 64), lambda i: (0, i, 0)),
                  pl.BlockSpec((R, 256), lambda i: (i, 0)),
                  pl.BlockSpec((R, 1), lambda i: (i, 0)),
                  _rep((1, 256)), _rep((256, 64))],
        out_specs=pl.BlockSpec((R, 64), lambda i: (i, 0)),
        out_shape=jax.ShapeDtypeStruct((N, 64), jnp.float32),
    )(S3, U3, dinv, b3.reshape(1, -1), W4)


def _tc_stage5(S4, U4, dinv, b4, W5):
    def body(s_ref, u_ref, dv_ref, b4_ref, w5_ref, out_ref):
        dv = dv_ref[...]
        y = (jnp.concatenate([s_ref[0], s_ref[1]], axis=1) + u_ref[...]) * dv
        h = jnp.maximum(y + b4_ref[...], 0.0)
        t = jnp.dot(h, w5_ref[...], preferred_element_type=jnp.float32)
        out_ref[...] = t * dv
    return pl.pallas_call(
        body,
        grid=(N // R,),
        in_specs=[pl.BlockSpec((2, R, 32), lambda i: (0, i, 0)),
                  pl.BlockSpec((R, 64), lambda i: (i, 0)),
                  pl.BlockSpec((R, 1), lambda i: (i, 0)),
                  _rep((1, 64)), _rep((64, 2))],
        out_specs=pl.BlockSpec((R, 2), lambda i: (i, 0)),
        out_shape=jax.ShapeDtypeStruct((N, 2), jnp.float32),
    )(S4, U4, dinv, b4.reshape(1, -1), W5)


def _tc_final(S5, U5, dinv, b5):
    def body(s_ref, u_ref, dv_ref, b5_ref, out_ref):
        y = (s_ref[0] + s_ref[1] + u_ref[...]) * dv_ref[...] + b5_ref[...]
        m = jnp.max(y, axis=1, keepdims=True)
        z = y - m
        lse = jnp.log(jnp.sum(jnp.exp(z), axis=1, keepdims=True))
        out_ref[...] = z - lse
    return pl.pallas_call(
        body,
        out_shape=jax.ShapeDtypeStruct((N, 2), jnp.float32),
    )(S5, U5, dinv, b5.reshape(1, -1))


def kernel(x, edge_index, W1, b1, W2, b2, W3, b3, W4, b4, W5, b5):
    ei = edge_index.astype(jnp.int32)
    src = ei[0]
    dst = ei[1]
    zdeg = jnp.zeros((ROWS_T,), jnp.float32)
    dst3e = dst.reshape(NC * NS, NCH_E, CHE)

    deg2 = _sc_degree(dst3e, zdeg)[:, :N]
    dinv, U1 = _tc_prolog(deg2.T, x)
    S1 = _sc_aggregate(U1, src, dst, 128, 4)
    U2 = _tc_stage2(S1, U1, dinv, W1, b1)
    S2 = _sc_aggregate(U2, src, dst, 256, 4)
    U3 = _tc_stage3(S2, U2, dinv, W2, b2, W3)
    S3 = _sc_aggregate(U3, src, dst, 256, 4)
    U4 = _tc_stage4(S3, U3, dinv, b3, W4)
    S4 = _sc_aggregate(U4, src, dst, 64, 2)
    U5 = _tc_stage5(S4, U4, dinv, b4, W5)
    S5 = _sc_aggregate(U5, src, dst, 2, 1)
    return _tc_final(S5, U5, dinv, b5)
